# baseline (speedup 1.0000x reference)
import functools

import jax
import jax.numpy as jnp
from jax import lax
from jax.experimental import pallas as pl
from jax.experimental.pallas import tpu as pltpu
from jax.experimental.pallas import tpu_sc as plsc

L = 16
PP = 32
JP = 32
SLOTS = PP * JP
GCH = SLOTS // 128


def _bc(s):
    return jnp.broadcast_to(s, (L,))


@functools.lru_cache(maxsize=None)
def _build(B, N, P, J):
    mesh = plsc.VectorSubcoreMesh(core_axis_name="c", subcore_axis_name="s")
    NC = 2
    TSL = J * PP

    @functools.partial(
        pl.kernel,
        mesh=mesh,
        out_type=jax.ShapeDtypeStruct((B, L), jnp.float32),
        compiler_params=pltpu.CompilerParams(needs_layout_passes=False),
        scratch_types=[
            pltpu.VMEM((GCH, 128), jnp.int32),
            pltpu.VMEM((SLOTS,), jnp.int32),
            pltpu.VMEM((SLOTS,), jnp.float32),
            pltpu.VMEM((TSL,), jnp.float32),
            pltpu.VMEM((TSL,), jnp.float32),
            pltpu.VMEM((L,), jnp.float32),
            pltpu.SemaphoreType.DMA,
        ],
    )
    def aeloss(tags_hbm, gidx_hbm, flg_hbm, out_hbm, gidx_v, flg_v, val_v,
               valt_v, wt_v, oval_v, sem):
        wid = lax.axis_index("s") * NC + lax.axis_index("c")
        zero = jnp.zeros((L,), jnp.float32)
        one = jnp.full((L,), 1.0, jnp.float32)
        lane = lax.iota(jnp.int32, L)

        pltpu.sync_copy(gidx_hbm.at[wid], gidx_v)
        pltpu.sync_copy(flg_hbm.at[wid], flg_v)
        copies = [
            pltpu.async_copy(
                tags_hbm.at[gidx_v.at[r]], val_v.at[pl.ds(r * 128, 128)], sem
            )
            for r in range(GCH)
        ]
        for cp in copies:
            cp.wait()

        lane32 = lane * PP
        cnt_lo = cnt_hi = sum_lo = sum_hi = zero
        for j in range(J):
            ilo = lane32 + j
            ihi = ilo + L * PP
            v_lo = plsc.load_gather(val_v, [ilo])
            v_hi = plsc.load_gather(val_v, [ihi])
            f_lo = plsc.load_gather(flg_v, [ilo])
            f_hi = plsc.load_gather(flg_v, [ihi])
            w_lo = jnp.where(f_lo > 0, one, zero)
            w_hi = jnp.where(f_hi > 0, one, zero)
            valt_v[pl.ds(j * PP, L)] = v_lo
            valt_v[pl.ds(j * PP + L, L)] = v_hi
            wt_v[pl.ds(j * PP, L)] = w_lo
            wt_v[pl.ds(j * PP + L, L)] = w_hi
            cnt_lo = cnt_lo + w_lo
            cnt_hi = cnt_hi + w_hi
            sum_lo = sum_lo + v_lo * w_lo
            sum_hi = sum_hi + v_hi * w_hi
        safe_lo = jnp.maximum(cnt_lo, one)
        safe_hi = jnp.maximum(cnt_hi, one)
        mean_lo = sum_lo / safe_lo
        mean_hi = sum_hi / safe_hi
        valid_lo = jnp.where(cnt_lo > 0, one, zero)
        valid_hi = jnp.where(cnt_hi > 0, one, zero)

        pacc_lo = pacc_hi = zero
        for j in range(J):
            w_lo = wt_v[pl.ds(j * PP, L)]
            w_hi = wt_v[pl.ds(j * PP + L, L)]
            d_lo = valt_v[pl.ds(j * PP, L)] - mean_lo
            d_hi = valt_v[pl.ds(j * PP + L, L)] - mean_hi
            pacc_lo = pacc_lo + d_lo * d_lo * w_lo
            pacc_hi = pacc_hi + d_hi * d_hi * w_hi
        pull_s = jnp.sum(pacc_lo / safe_lo * valid_lo) + jnp.sum(
            pacc_hi / safe_hi * valid_hi)
        ntags = _bc(jnp.sum(valid_lo) + jnp.sum(valid_hi))

        acc_lo = acc_hi = zero
        for i in range(P):
            sel = lane == (i % L)
            src_m = mean_lo if i < L else mean_hi
            src_v = valid_lo if i < L else valid_hi
            m_i = _bc(jnp.sum(jnp.where(sel, src_m, zero)))
            v_i = _bc(jnp.sum(jnp.where(sel, src_v, zero)))
            d_lo = m_i - mean_lo
            d_hi = m_i - mean_hi
            acc_lo = acc_lo + v_i * jnp.exp(-(d_lo * d_lo)) * valid_lo
            acc_hi = acc_hi + v_i * jnp.exp(-(d_hi * d_hi)) * valid_hi
        push_tot = _bc(jnp.sum(acc_lo) + jnp.sum(acc_hi)) - ntags
        denom = jnp.maximum(ntags * (ntags - one), one)
        push = 0.5 * push_tot / denom
        pull = _bc(pull_s) / jnp.maximum(ntags, one)

        oval_v[...] = jnp.where(lane == 0, pull, jnp.where(lane == 1, push, zero))
        pltpu.sync_copy(oval_v, out_hbm.at[wid])

    return aeloss


def kernel(input, input1):
    tags = input
    keypoints = input1
    B, N, D = tags.shape
    P, J = keypoints.shape[1], keypoints.shape[2]

    idx = keypoints[..., 0] + (jnp.arange(B, dtype=jnp.int32) * N)[:, None, None]
    flg = keypoints[..., 1]
    gidx = jnp.zeros((B, PP, JP), jnp.int32).at[:, :P, :J].set(idx)
    flgp = jnp.zeros((B, PP, JP), jnp.int32).at[:, :P, :J].set(flg)

    out = _build(B, N, P, J)(
        tags.reshape(B * N),
        gidx.reshape(B, GCH, 128),
        flgp.reshape(B, SLOTS),
    )
    return out[:, :2]


# speedup vs baseline: 2.3615x; 2.3615x over previous
import functools

import jax
import jax.numpy as jnp
from jax import lax
from jax.experimental import pallas as pl
from jax.experimental.pallas import tpu as pltpu
from jax.experimental.pallas import tpu_sc as plsc

L = 16
PP = 32
JP = 20
SLOTS = JP * PP
GCH = SLOTS // 128


def _bc(s):
    return jnp.broadcast_to(s, (L,))


@functools.lru_cache(maxsize=None)
def _build(B, N, P, J):
    mesh = plsc.VectorSubcoreMesh(core_axis_name="c", subcore_axis_name="s")
    NC = 2

    @functools.partial(
        pl.kernel,
        mesh=mesh,
        out_type=jax.ShapeDtypeStruct((B, L), jnp.float32),
        compiler_params=pltpu.CompilerParams(needs_layout_passes=False),
        scratch_types=[
            pltpu.VMEM((GCH, 128), jnp.int32),
            pltpu.VMEM((SLOTS,), jnp.int32),
            pltpu.VMEM((SLOTS,), jnp.float32),
            pltpu.VMEM((L,), jnp.float32),
            pltpu.SemaphoreType.DMA,
        ],
    )
    def aeloss(tags_hbm, gidx_hbm, flg_hbm, out_hbm, gidx_v, flg_v, val_v,
               oval_v, sem):
        wid = lax.axis_index("s") * NC + lax.axis_index("c")
        zero = jnp.zeros((L,), jnp.float32)
        one = jnp.full((L,), 1.0, jnp.float32)
        lane = lax.iota(jnp.int32, L)

        pltpu.sync_copy(gidx_hbm.at[wid], gidx_v)
        pltpu.sync_copy(flg_hbm.at[wid], flg_v)
        copies = [
            pltpu.async_copy(
                tags_hbm.at[gidx_v.at[r]], val_v.at[pl.ds(r * 128, 128)], sem
            )
            for r in range(GCH)
        ]
        for cp in copies:
            cp.wait()

        cnt_lo = cnt_hi = sum_lo = sum_hi = zero
        for j in range(J):
            f_lo = flg_v[pl.ds(j * PP, L)]
            f_hi = flg_v[pl.ds(j * PP + L, L)]
            v_lo = val_v[pl.ds(j * PP, L)]
            v_hi = val_v[pl.ds(j * PP + L, L)]
            w_lo = jnp.where(f_lo > 0, one, zero)
            w_hi = jnp.where(f_hi > 0, one, zero)
            cnt_lo = cnt_lo + w_lo
            cnt_hi = cnt_hi + w_hi
            sum_lo = sum_lo + v_lo * w_lo
            sum_hi = sum_hi + v_hi * w_hi
        safe_lo = jnp.maximum(cnt_lo, one)
        safe_hi = jnp.maximum(cnt_hi, one)
        mean_lo = sum_lo / safe_lo
        mean_hi = sum_hi / safe_hi
        valid_lo = jnp.where(cnt_lo > 0, one, zero)
        valid_hi = jnp.where(cnt_hi > 0, one, zero)

        pacc_lo = pacc_hi = zero
        for j in range(J):
            f_lo = flg_v[pl.ds(j * PP, L)]
            f_hi = flg_v[pl.ds(j * PP + L, L)]
            w_lo = jnp.where(f_lo > 0, one, zero)
            w_hi = jnp.where(f_hi > 0, one, zero)
            d_lo = val_v[pl.ds(j * PP, L)] - mean_lo
            d_hi = val_v[pl.ds(j * PP + L, L)] - mean_hi
            pacc_lo = pacc_lo + d_lo * d_lo * w_lo
            pacc_hi = pacc_hi + d_hi * d_hi * w_hi
        pull_s = jnp.sum(pacc_lo / safe_lo * valid_lo) + jnp.sum(
            pacc_hi / safe_hi * valid_hi)
        ntags = _bc(jnp.sum(valid_lo) + jnp.sum(valid_hi))

        acc_lo = acc_hi = zero
        for i in range(P):
            sel = lane == (i % L)
            src_m = mean_lo if i < L else mean_hi
            src_v = valid_lo if i < L else valid_hi
            m_i = _bc(jnp.sum(jnp.where(sel, src_m, zero)))
            v_i = _bc(jnp.sum(jnp.where(sel, src_v, zero)))
            d_lo = m_i - mean_lo
            d_hi = m_i - mean_hi
            acc_lo = acc_lo + v_i * jnp.exp(-(d_lo * d_lo)) * valid_lo
            acc_hi = acc_hi + v_i * jnp.exp(-(d_hi * d_hi)) * valid_hi
        push_tot = _bc(jnp.sum(acc_lo) + jnp.sum(acc_hi)) - ntags
        denom = jnp.maximum(ntags * (ntags - one), one)
        push = 0.5 * push_tot / denom
        pull = _bc(pull_s) / jnp.maximum(ntags, one)

        oval_v[...] = jnp.where(lane == 0, pull, jnp.where(lane == 1, push, zero))
        pltpu.sync_copy(oval_v, out_hbm.at[wid])

    return aeloss


def kernel(input, input1):
    tags = input
    keypoints = input1
    B, N, D = tags.shape
    P, J = keypoints.shape[1], keypoints.shape[2]

    idx = keypoints[..., 0] + (jnp.arange(B, dtype=jnp.int32) * N)[:, None, None]
    flg = keypoints[..., 1]
    idx_t = idx.transpose(0, 2, 1)
    flg_t = flg.transpose(0, 2, 1)
    gidx = jnp.zeros((B, JP, PP), jnp.int32).at[:, :J, :P].set(idx_t)
    flgp = jnp.zeros((B, JP, PP), jnp.int32).at[:, :J, :P].set(flg_t)

    out = _build(B, N, P, J)(
        tags.reshape(B * N),
        gidx.reshape(B, GCH, 128),
        flgp.reshape(B, SLOTS),
    )
    return out[:, :2]


# speedup vs baseline: 2.3638x; 1.0009x over previous
import functools

import jax
import jax.numpy as jnp
from jax import lax
from jax.experimental import pallas as pl
from jax.experimental.pallas import tpu as pltpu
from jax.experimental.pallas import tpu_sc as plsc

L = 16
PP = 32
JP = 20
SLOTS = JP * PP
GCH = SLOTS // 128


def _bc(s):
    return jnp.broadcast_to(s, (L,))


@functools.lru_cache(maxsize=None)
def _build(B, N, P, J):
    mesh = plsc.VectorSubcoreMesh(core_axis_name="c", subcore_axis_name="s")
    NC = 2

    @functools.partial(
        pl.kernel,
        mesh=mesh,
        out_type=jax.ShapeDtypeStruct((B, L), jnp.float32),
        compiler_params=pltpu.CompilerParams(needs_layout_passes=False),
        scratch_types=[
            pltpu.VMEM((GCH, 128), jnp.int32),
            pltpu.VMEM((SLOTS,), jnp.int32),
            pltpu.VMEM((SLOTS,), jnp.float32),
            pltpu.VMEM((L,), jnp.float32),
            pltpu.SemaphoreType.DMA,
        ],
    )
    def aeloss(tags_hbm, gidx_hbm, flg_hbm, out_hbm, gidx_v, flg_v, val_v,
               oval_v, sem):
        wid = lax.axis_index("s") * NC + lax.axis_index("c")
        zero = jnp.zeros((L,), jnp.float32)
        one = jnp.full((L,), 1.0, jnp.float32)
        lane = lax.iota(jnp.int32, L)

        with jax.named_scope("stage"):
            pltpu.sync_copy(gidx_hbm.at[wid], gidx_v)
            pltpu.sync_copy(flg_hbm.at[wid], flg_v)
        with jax.named_scope("fire"):
            copies = [
                pltpu.async_copy(
                    tags_hbm.at[gidx_v.at[r]], val_v.at[pl.ds(r * 128, 128)], sem
                )
                for r in range(GCH)
            ]
        with jax.named_scope("drain"):
            for cp in copies:
                cp.wait()

        scope_a = jax.named_scope("passA")
        scope_a.__enter__()
        cnt_lo = cnt_hi = sum_lo = sum_hi = zero
        for j in range(J):
            f_lo = flg_v[pl.ds(j * PP, L)]
            f_hi = flg_v[pl.ds(j * PP + L, L)]
            v_lo = val_v[pl.ds(j * PP, L)]
            v_hi = val_v[pl.ds(j * PP + L, L)]
            w_lo = jnp.where(f_lo > 0, one, zero)
            w_hi = jnp.where(f_hi > 0, one, zero)
            cnt_lo = cnt_lo + w_lo
            cnt_hi = cnt_hi + w_hi
            sum_lo = sum_lo + v_lo * w_lo
            sum_hi = sum_hi + v_hi * w_hi
        safe_lo = jnp.maximum(cnt_lo, one)
        safe_hi = jnp.maximum(cnt_hi, one)
        mean_lo = sum_lo / safe_lo
        mean_hi = sum_hi / safe_hi
        valid_lo = jnp.where(cnt_lo > 0, one, zero)
        valid_hi = jnp.where(cnt_hi > 0, one, zero)
        scope_a.__exit__(None, None, None)

        scope_b = jax.named_scope("passB")
        scope_b.__enter__()
        pacc_lo = pacc_hi = zero
        for j in range(J):
            f_lo = flg_v[pl.ds(j * PP, L)]
            f_hi = flg_v[pl.ds(j * PP + L, L)]
            w_lo = jnp.where(f_lo > 0, one, zero)
            w_hi = jnp.where(f_hi > 0, one, zero)
            d_lo = val_v[pl.ds(j * PP, L)] - mean_lo
            d_hi = val_v[pl.ds(j * PP + L, L)] - mean_hi
            pacc_lo = pacc_lo + d_lo * d_lo * w_lo
            pacc_hi = pacc_hi + d_hi * d_hi * w_hi
        pull_s = jnp.sum(pacc_lo / safe_lo * valid_lo) + jnp.sum(
            pacc_hi / safe_hi * valid_hi)
        ntags = _bc(jnp.sum(valid_lo) + jnp.sum(valid_hi))
        scope_b.__exit__(None, None, None)

        scope_p = jax.named_scope("push")
        scope_p.__enter__()
        acc_lo = acc_hi = zero
        for i in range(P):
            sel = lane == (i % L)
            src_m = mean_lo if i < L else mean_hi
            src_v = valid_lo if i < L else valid_hi
            m_i = _bc(jnp.sum(jnp.where(sel, src_m, zero)))
            v_i = _bc(jnp.sum(jnp.where(sel, src_v, zero)))
            d_lo = m_i - mean_lo
            d_hi = m_i - mean_hi
            acc_lo = acc_lo + v_i * jnp.exp(-(d_lo * d_lo)) * valid_lo
            acc_hi = acc_hi + v_i * jnp.exp(-(d_hi * d_hi)) * valid_hi
        push_tot = _bc(jnp.sum(acc_lo) + jnp.sum(acc_hi)) - ntags
        denom = jnp.maximum(ntags * (ntags - one), one)
        push = 0.5 * push_tot / denom
        pull = _bc(pull_s) / jnp.maximum(ntags, one)
        scope_p.__exit__(None, None, None)

        with jax.named_scope("out"):
            oval_v[...] = jnp.where(
                lane == 0, pull, jnp.where(lane == 1, push, zero))
            pltpu.sync_copy(oval_v, out_hbm.at[wid])

    return aeloss


def kernel(input, input1):
    tags = input
    keypoints = input1
    B, N, D = tags.shape
    P, J = keypoints.shape[1], keypoints.shape[2]

    idx = keypoints[..., 0] + (jnp.arange(B, dtype=jnp.int32) * N)[:, None, None]
    flg = keypoints[..., 1]
    idx_t = idx.transpose(0, 2, 1)
    flg_t = flg.transpose(0, 2, 1)
    gidx = jnp.zeros((B, JP, PP), jnp.int32).at[:, :J, :P].set(idx_t)
    flgp = jnp.zeros((B, JP, PP), jnp.int32).at[:, :J, :P].set(flg_t)

    out = _build(B, N, P, J)(
        tags.reshape(B * N),
        gidx.reshape(B, GCH, 128),
        flgp.reshape(B, SLOTS),
    )
    return out[:, :2]


# speedup vs baseline: 2.3775x; 1.0058x over previous
import functools

import jax
import jax.numpy as jnp
from jax import lax
from jax.experimental import pallas as pl
from jax.experimental.pallas import tpu as pltpu
from jax.experimental.pallas import tpu_sc as plsc

L = 16
PP = 32
JP = 20
SLOTS = JP * PP
GCH = 16
GW = SLOTS // GCH


def _bc(s):
    return jnp.broadcast_to(s, (L,))


@functools.lru_cache(maxsize=None)
def _build(B, N, P, J):
    mesh = plsc.VectorSubcoreMesh(core_axis_name="c", subcore_axis_name="s")
    NC = 2

    @functools.partial(
        pl.kernel,
        mesh=mesh,
        out_type=jax.ShapeDtypeStruct((B, L), jnp.float32),
        compiler_params=pltpu.CompilerParams(needs_layout_passes=False),
        scratch_types=[
            pltpu.VMEM((GCH, GW), jnp.int32),
            pltpu.VMEM((SLOTS,), jnp.int32),
            pltpu.VMEM((SLOTS,), jnp.float32),
            pltpu.VMEM((L,), jnp.float32),
            pltpu.SemaphoreType.DMA,
        ],
    )
    def aeloss(tags_hbm, gidx_hbm, flg_hbm, out_hbm, gidx_v, flg_v, val_v,
               oval_v, sem):
        wid = lax.axis_index("s") * NC + lax.axis_index("c")
        zero = jnp.zeros((L,), jnp.float32)
        one = jnp.full((L,), 1.0, jnp.float32)
        lane = lax.iota(jnp.int32, L)

        with jax.named_scope("stage"):
            pltpu.sync_copy(gidx_hbm.at[wid], gidx_v)
        with jax.named_scope("fire"):
            copies = [
                pltpu.async_copy(
                    tags_hbm.at[gidx_v.at[r]], val_v.at[pl.ds(r * GW, GW)], sem
                )
                for r in range(GCH)
            ]
        with jax.named_scope("stage2"):
            pltpu.sync_copy(flg_hbm.at[wid], flg_v)
        with jax.named_scope("drain"):
            for cp in copies:
                cp.wait()

        scope_a = jax.named_scope("passA")
        scope_a.__enter__()
        cnt_lo = cnt_hi = sum_lo = sum_hi = zero
        for j in range(J):
            f_lo = flg_v[pl.ds(j * PP, L)]
            f_hi = flg_v[pl.ds(j * PP + L, L)]
            v_lo = val_v[pl.ds(j * PP, L)]
            v_hi = val_v[pl.ds(j * PP + L, L)]
            w_lo = jnp.where(f_lo > 0, one, zero)
            w_hi = jnp.where(f_hi > 0, one, zero)
            cnt_lo = cnt_lo + w_lo
            cnt_hi = cnt_hi + w_hi
            sum_lo = sum_lo + v_lo * w_lo
            sum_hi = sum_hi + v_hi * w_hi
        safe_lo = jnp.maximum(cnt_lo, one)
        safe_hi = jnp.maximum(cnt_hi, one)
        mean_lo = sum_lo / safe_lo
        mean_hi = sum_hi / safe_hi
        valid_lo = jnp.where(cnt_lo > 0, one, zero)
        valid_hi = jnp.where(cnt_hi > 0, one, zero)
        scope_a.__exit__(None, None, None)

        scope_b = jax.named_scope("passB")
        scope_b.__enter__()
        pacc_lo = pacc_hi = zero
        for j in range(J):
            f_lo = flg_v[pl.ds(j * PP, L)]
            f_hi = flg_v[pl.ds(j * PP + L, L)]
            w_lo = jnp.where(f_lo > 0, one, zero)
            w_hi = jnp.where(f_hi > 0, one, zero)
            d_lo = val_v[pl.ds(j * PP, L)] - mean_lo
            d_hi = val_v[pl.ds(j * PP + L, L)] - mean_hi
            pacc_lo = pacc_lo + d_lo * d_lo * w_lo
            pacc_hi = pacc_hi + d_hi * d_hi * w_hi
        pull_s = jnp.sum(pacc_lo / safe_lo * valid_lo) + jnp.sum(
            pacc_hi / safe_hi * valid_hi)
        ntags = _bc(jnp.sum(valid_lo) + jnp.sum(valid_hi))
        scope_b.__exit__(None, None, None)

        scope_p = jax.named_scope("push")
        scope_p.__enter__()
        acc_lo = acc_hi = zero
        for i in range(P):
            sel = lane == (i % L)
            src_m = mean_lo if i < L else mean_hi
            src_v = valid_lo if i < L else valid_hi
            m_i = _bc(jnp.sum(jnp.where(sel, src_m, zero)))
            v_i = _bc(jnp.sum(jnp.where(sel, src_v, zero)))
            d_lo = m_i - mean_lo
            d_hi = m_i - mean_hi
            acc_lo = acc_lo + v_i * jnp.exp(-(d_lo * d_lo)) * valid_lo
            acc_hi = acc_hi + v_i * jnp.exp(-(d_hi * d_hi)) * valid_hi
        push_tot = _bc(jnp.sum(acc_lo) + jnp.sum(acc_hi)) - ntags
        denom = jnp.maximum(ntags * (ntags - one), one)
        push = 0.5 * push_tot / denom
        pull = _bc(pull_s) / jnp.maximum(ntags, one)
        scope_p.__exit__(None, None, None)

        with jax.named_scope("out"):
            oval_v[...] = jnp.where(
                lane == 0, pull, jnp.where(lane == 1, push, zero))
            pltpu.sync_copy(oval_v, out_hbm.at[wid])

    return aeloss


def kernel(input, input1):
    tags = input
    keypoints = input1
    B, N, D = tags.shape
    P, J = keypoints.shape[1], keypoints.shape[2]

    idx = keypoints[..., 0] + (jnp.arange(B, dtype=jnp.int32) * N)[:, None, None]
    flg = keypoints[..., 1]
    idx_t = idx.transpose(0, 2, 1)
    flg_t = flg.transpose(0, 2, 1)
    gidx = jnp.zeros((B, JP, PP), jnp.int32).at[:, :J, :P].set(idx_t)
    flgp = jnp.zeros((B, JP, PP), jnp.int32).at[:, :J, :P].set(flg_t)

    out = _build(B, N, P, J)(
        tags.reshape(B * N),
        gidx.reshape(B, GCH, GW),
        flgp.reshape(B, SLOTS),
    )
    return out[:, :2]


# speedup vs baseline: 3.2643x; 1.3730x over previous
import functools

import jax
import jax.numpy as jnp
from jax import lax
from jax.experimental import pallas as pl
from jax.experimental.pallas import tpu as pltpu
from jax.experimental.pallas import tpu_sc as plsc

L = 16
PP = 32
JP = 32
SLOTS = PP * JP
GCH = SLOTS // 128


def _bc(s):
    return jnp.broadcast_to(s, (L,))


@functools.lru_cache(maxsize=None)
def _build(B, N, P, J):
    mesh = plsc.VectorSubcoreMesh(core_axis_name="c", subcore_axis_name="s")
    NC = 2

    @functools.partial(
        pl.kernel,
        mesh=mesh,
        out_type=jax.ShapeDtypeStruct((B, L), jnp.float32),
        compiler_params=pltpu.CompilerParams(needs_layout_passes=False),
        scratch_types=[
            pltpu.VMEM((GCH, 128), jnp.int32),
            pltpu.VMEM((SLOTS,), jnp.int32),
            pltpu.VMEM((SLOTS,), jnp.float32),
            pltpu.VMEM((L,), jnp.float32),
            pltpu.SemaphoreType.DMA,
        ],
    )
    def aeloss(tags_hbm, idx_hbm, flg_hbm, out_hbm, idx_v, flg_v, val_v, out_v, sem):
        wid = lax.axis_index("s") * NC + lax.axis_index("c")

        with jax.named_scope("stage"):
            pltpu.sync_copy(idx_hbm.at[wid], idx_v)
            pltpu.sync_copy(flg_hbm.at[wid], flg_v)

        with jax.named_scope("rmw"):
            off = _bc(wid * N).astype(jnp.int32)
            for j in range(GCH):
                for c in range(128 // L):
                    sl = idx_v[j, pl.ds(c * L, L)]
                    idx_v[j, pl.ds(c * L, L)] = sl + off

        with jax.named_scope("fire"):
            copies = [
                pltpu.async_copy(
                    tags_hbm.at[idx_v.at[j]], val_v.at[pl.ds(j * 128, 128)], sem
                )
                for j in range(GCH)
            ]
        with jax.named_scope("drain"):
            for cp in copies:
                cp.wait()

        zero = jnp.zeros((L,), jnp.float32)
        one = jnp.full((L,), 1.0, jnp.float32)
        lane = lax.iota(jnp.int32, L)

        def person_stats(p):
            base = p * JP
            v0 = val_v[pl.ds(base, L)]
            v1 = val_v[pl.ds(base + L, L)]
            f0 = flg_v[pl.ds(base, L)]
            f1 = flg_v[pl.ds(base + L, L)]
            vis0 = jnp.where(f0 > 0, one, zero)
            vis1 = jnp.where(f1 > 0, one, zero)
            cnt = _bc(jnp.sum(vis0 + vis1))
            safe = jnp.maximum(cnt, one)
            mean = _bc(jnp.sum(v0 * vis0 + v1 * vis1)) / safe
            valid = jnp.where(cnt > 0, one, zero)
            return v0, v1, vis0, vis1, mean, valid, safe

        means_lo = zero
        means_hi = zero
        valid_lo = zero
        valid_hi = zero
        pull_acc = zero
        ntags = zero
        for p in range(P):
            v0, v1, vis0, vis1, mean, valid, safe = person_stats(p)
            d0 = v0 - mean
            d1 = v1 - mean
            pp = _bc(jnp.sum(d0 * d0 * vis0 + d1 * d1 * vis1))
            pull_acc = pull_acc + pp / safe * valid
            ntags = ntags + valid
            if p < L:
                sel = lane == p
                means_lo = jnp.where(sel, mean, means_lo)
                valid_lo = jnp.where(sel, valid, valid_lo)
            else:
                sel = lane == (p - L)
                means_hi = jnp.where(sel, mean, means_hi)
                valid_hi = jnp.where(sel, valid, valid_hi)

        acc_lo = zero
        acc_hi = zero
        for p in range(P):
            _, _, _, _, mean_i, valid_i, _ = person_stats(p)
            dlo = mean_i - means_lo
            dhi = mean_i - means_hi
            acc_lo = acc_lo + valid_i * jnp.exp(-(dlo * dlo)) * valid_lo
            acc_hi = acc_hi + valid_i * jnp.exp(-(dhi * dhi)) * valid_hi

        push_tot = _bc(jnp.sum(acc_lo) + jnp.sum(acc_hi)) - ntags
        denom = jnp.maximum(ntags * (ntags - one), one)
        push = 0.5 * push_tot / denom
        pull = pull_acc / jnp.maximum(ntags, one)

        out_v[...] = jnp.where(lane == 0, pull, jnp.where(lane == 1, push, zero))
        pltpu.sync_copy(out_v, out_hbm.at[wid])

    return aeloss


def kernel(input, input1):
    tags = input
    keypoints = input1
    B, N, D = tags.shape
    P, J = keypoints.shape[1], keypoints.shape[2]

    idx = keypoints[..., 0]
    flg = keypoints[..., 1]
    idx_pad = jnp.zeros((B, PP, JP), jnp.int32).at[:, :P, :J].set(idx)
    flg_pad = jnp.zeros((B, PP, JP), jnp.int32).at[:, :P, :J].set(flg)

    out16 = _build(B, N, P, J)(
        tags.reshape(B * N),
        idx_pad.reshape(B, GCH, 128),
        flg_pad.reshape(B, SLOTS),
    )
    return out16[:, :2]


# speedup vs baseline: 3.5644x; 1.0919x over previous
import functools

import jax
import jax.numpy as jnp
from jax import lax
from jax.experimental import pallas as pl
from jax.experimental.pallas import tpu as pltpu
from jax.experimental.pallas import tpu_sc as plsc

L = 16
PP = 32
JP = 20
SLOTS = JP * PP
GCH = 16
GW = SLOTS // GCH


def _bc(s):
    return jnp.broadcast_to(s, (L,))


@functools.lru_cache(maxsize=None)
def _build(B, N, P, J):
    mesh = plsc.VectorSubcoreMesh(core_axis_name="c", subcore_axis_name="s")
    NC = 2

    @functools.partial(
        pl.kernel,
        mesh=mesh,
        out_type=jax.ShapeDtypeStruct((B, L), jnp.float32),
        compiler_params=pltpu.CompilerParams(needs_layout_passes=False),
        scratch_types=[
            pltpu.VMEM((GCH, GW), jnp.int32),
            pltpu.VMEM((SLOTS,), jnp.int32),
            pltpu.VMEM((SLOTS,), jnp.float32),
            pltpu.VMEM((L,), jnp.float32),
            pltpu.SemaphoreType.DMA,
        ],
    )
    def aeloss(tags_hbm, gidx_hbm, flg_hbm, out_hbm, gidx_v, flg_v, val_v,
               oval_v, sem):
        wid = lax.axis_index("s") * NC + lax.axis_index("c")
        zero = jnp.zeros((L,), jnp.float32)
        one = jnp.full((L,), 1.0, jnp.float32)
        lane = lax.iota(jnp.int32, L)

        with jax.named_scope("stage"):
            pltpu.sync_copy(gidx_hbm.at[wid], gidx_v)
        with jax.named_scope("fire"):
            copies = [
                pltpu.async_copy(
                    tags_hbm.at[gidx_v.at[r]], val_v.at[pl.ds(r * GW, GW)], sem
                )
                for r in range(GCH)
            ]
        with jax.named_scope("stage2"):
            pltpu.sync_copy(flg_hbm.at[wid], flg_v)
        with jax.named_scope("drain"):
            for cp in copies:
                cp.wait()

        scope_a = jax.named_scope("passA")
        scope_a.__enter__()
        cnt_lo = cnt_hi = sum_lo = sum_hi = zero
        for j in range(J):
            f_lo = flg_v[pl.ds(j * PP, L)]
            f_hi = flg_v[pl.ds(j * PP + L, L)]
            v_lo = val_v[pl.ds(j * PP, L)]
            v_hi = val_v[pl.ds(j * PP + L, L)]
            w_lo = jnp.where(f_lo > 0, one, zero)
            w_hi = jnp.where(f_hi > 0, one, zero)
            cnt_lo = cnt_lo + w_lo
            cnt_hi = cnt_hi + w_hi
            sum_lo = sum_lo + v_lo * w_lo
            sum_hi = sum_hi + v_hi * w_hi
        safe_lo = jnp.maximum(cnt_lo, one)
        safe_hi = jnp.maximum(cnt_hi, one)
        mean_lo = sum_lo / safe_lo
        mean_hi = sum_hi / safe_hi
        valid_lo = jnp.where(cnt_lo > 0, one, zero)
        valid_hi = jnp.where(cnt_hi > 0, one, zero)
        scope_a.__exit__(None, None, None)

        scope_b = jax.named_scope("passB")
        scope_b.__enter__()
        pacc_lo = pacc_hi = zero
        for j in range(J):
            f_lo = flg_v[pl.ds(j * PP, L)]
            f_hi = flg_v[pl.ds(j * PP + L, L)]
            w_lo = jnp.where(f_lo > 0, one, zero)
            w_hi = jnp.where(f_hi > 0, one, zero)
            d_lo = val_v[pl.ds(j * PP, L)] - mean_lo
            d_hi = val_v[pl.ds(j * PP + L, L)] - mean_hi
            pacc_lo = pacc_lo + d_lo * d_lo * w_lo
            pacc_hi = pacc_hi + d_hi * d_hi * w_hi
        pull_s = jnp.sum(pacc_lo / safe_lo * valid_lo) + jnp.sum(
            pacc_hi / safe_hi * valid_hi)
        ntags = _bc(jnp.sum(valid_lo) + jnp.sum(valid_hi))
        scope_b.__exit__(None, None, None)

        scope_p = jax.named_scope("push")
        scope_p.__enter__()
        acc_lo = acc_hi = zero
        for i in range(P):
            sel = lane == (i % L)
            src_m = mean_lo if i < L else mean_hi
            src_v = valid_lo if i < L else valid_hi
            m_i = _bc(jnp.sum(jnp.where(sel, src_m, zero)))
            v_i = _bc(jnp.sum(jnp.where(sel, src_v, zero)))
            d_lo = m_i - mean_lo
            d_hi = m_i - mean_hi
            acc_lo = acc_lo + v_i * jnp.exp(-(d_lo * d_lo)) * valid_lo
            acc_hi = acc_hi + v_i * jnp.exp(-(d_hi * d_hi)) * valid_hi
        push_tot = _bc(jnp.sum(acc_lo) + jnp.sum(acc_hi)) - ntags
        denom = jnp.maximum(ntags * (ntags - one), one)
        push = 0.5 * push_tot / denom
        pull = _bc(pull_s) / jnp.maximum(ntags, one)
        scope_p.__exit__(None, None, None)

        with jax.named_scope("out"):
            oval_v[...] = jnp.where(
                lane == 0, pull, jnp.where(lane == 1, push, zero))
            pltpu.sync_copy(oval_v, out_hbm.at[wid])

    return aeloss


def kernel(input, input1):
    tags = input
    keypoints = input1
    B, N, D = tags.shape
    P, J = keypoints.shape[1], keypoints.shape[2]

    idx_t = keypoints[..., 0].transpose(0, 2, 1)
    flg_t = keypoints[..., 1].transpose(0, 2, 1)
    gidx = (jnp.zeros((B, JP, PP), jnp.int32).at[:, :J, :P].set(idx_t)
            + (jnp.arange(B, dtype=jnp.int32) * N)[:, None, None])
    flgp = jnp.zeros((B, JP, PP), jnp.int32).at[:, :J, :P].set(flg_t)

    out = _build(B, N, P, J)(
        tags.reshape(B * N),
        gidx.reshape(B, GCH, GW),
        flgp.reshape(B, SLOTS),
    )
    return out[:, :2]
